# baseline (speedup 1.0000x reference)
import jax
import jax.numpy as jnp
from jax import lax
from jax.experimental import pallas as pl
from jax.experimental.pallas import tpu as pltpu

B, S, D, O = 8, 2048, 1024, 1024
BM = 512


def _mm_body(x_ref, w_ref, b_ref, m_ref, o_ref, wb_ref):
    @pl.when(pl.program_id(0) == 0)
    def _():
        wb_ref[...] = w_ref[...].astype(jnp.bfloat16)

    xb = x_ref[...].astype(jnp.bfloat16)
    acc = lax.dot_general(
        xb, wb_ref[...],
        dimension_numbers=(((1,), (1,)), ((), ())),
        preferred_element_type=jnp.float32,
    )
    mf = m_ref[...].astype(jnp.float32)
    o_ref[...] = (acc + b_ref[...]) * mf


def kernel(input, data_mask, W, b):
    x = input.reshape(B * S, D)
    m2 = data_mask.reshape(B * S, 1)
    b2 = b.reshape(1, O)

    grid = (B * S // BM,)
    out = pl.pallas_call(
        _mm_body,
        grid=grid,
        in_specs=[
            pl.BlockSpec((BM, D), lambda i: (i, 0)),
            pl.BlockSpec((O, D), lambda i: (0, 0)),
            pl.BlockSpec((1, O), lambda i: (0, 0)),
            pl.BlockSpec((BM, 1), lambda i: (i, 0)),
        ],
        out_specs=pl.BlockSpec((BM, O), lambda i: (i, 0)),
        out_shape=jax.ShapeDtypeStruct((B * S, O), jnp.float32),
        scratch_shapes=[pltpu.VMEM((O, D), jnp.bfloat16)],
        compiler_params=pltpu.CompilerParams(
            dimension_semantics=("arbitrary",),
        ),
    )(x, W, b2, m2)
    return out.reshape(B, S, O)


# speedup vs baseline: 1.2344x; 1.2344x over previous
import jax
import jax.numpy as jnp
from jax import lax
from jax.experimental import pallas as pl
from jax.experimental.pallas import tpu as pltpu

B, S, D, O = 8, 2048, 1024, 1024
BM = 2048


def _mm_body(x_ref, w_ref, b_ref, m_ref, o_ref, wb_ref):
    @pl.when(pl.program_id(0) == 0)
    def _():
        wb_ref[...] = w_ref[...].astype(jnp.bfloat16)

    xb = x_ref[...].astype(jnp.bfloat16)
    acc = lax.dot_general(
        xb, wb_ref[...],
        dimension_numbers=(((1,), (1,)), ((), ())),
        preferred_element_type=jnp.float32,
    )
    mf = m_ref[...].astype(jnp.float32)
    o_ref[...] = (acc + b_ref[...]) * mf


def kernel(input, data_mask, W, b):
    x = input.reshape(B * S, D)
    m2 = data_mask.reshape(B * S, 1)
    b2 = b.reshape(1, O)

    grid = (B * S // BM,)
    out = pl.pallas_call(
        _mm_body,
        grid=grid,
        in_specs=[
            pl.BlockSpec((BM, D), lambda i: (i, 0)),
            pl.BlockSpec((O, D), lambda i: (0, 0)),
            pl.BlockSpec((1, O), lambda i: (0, 0)),
            pl.BlockSpec((BM, 1), lambda i: (i, 0)),
        ],
        out_specs=pl.BlockSpec((BM, O), lambda i: (i, 0)),
        out_shape=jax.ShapeDtypeStruct((B * S, O), jnp.float32),
        scratch_shapes=[pltpu.VMEM((O, D), jnp.bfloat16)],
        compiler_params=pltpu.CompilerParams(
            dimension_semantics=("arbitrary",),
        ),
    )(x, W, b2, m2)
    return out.reshape(B, S, O)
